# Initial kernel scaffold; baseline (speedup 1.0000x reference)
import jax
import jax.numpy as jnp
from jax.experimental import pallas as pl


def kernel(hidden_states, router_w, ws, w2s):
    raise NotImplementedError("write your pallas kernel here")


# speedup vs baseline: 1.1793x; 1.1793x over previous
import functools

import jax
import jax.numpy as jnp
from jax.experimental import pallas as pl
from jax.experimental.pallas import tpu as pltpu

T = 2048
D = 1024
I = 4096
E = 8
K = 2

B = 256
BI = 512
G = (K * T) // B + (E - 1)
P = G * B
NJ = I // BI


def _router_body(x_ref, rw_ref, w_ref, i_ref):
    logits = jax.lax.dot_general(
        x_ref[...], rw_ref[...], (((1,), (1,)), ((), ())),
        preferred_element_type=jnp.float32)
    col = jax.lax.broadcasted_iota(jnp.int32, logits.shape, 1)
    m1 = jnp.max(logits, axis=-1, keepdims=True)
    i1 = jnp.min(jnp.where(logits == m1, col, E), axis=-1, keepdims=True)
    masked = jnp.where(col == i1, -jnp.inf, logits)
    m2 = jnp.max(masked, axis=-1, keepdims=True)
    i2 = jnp.min(jnp.where(masked == m2, col, E), axis=-1, keepdims=True)
    w1 = 1.0 / (1.0 + jnp.exp(m2 - m1))
    w2 = 1.0 - w1
    w_ref[...] = jnp.where(col == 0, w1, jnp.where(col == 1, w2, 0.0))
    i_ref[...] = jnp.where(col == 0, i1, jnp.where(col == 1, i2, 0))


def _router(x, router_w):
    return pl.pallas_call(
        _router_body,
        out_shape=(
            jax.ShapeDtypeStruct((T, E), jnp.float32),
            jax.ShapeDtypeStruct((T, E), jnp.int32),
        ),
    )(x, router_w)


def _moe_body(be_ref, xs_ref, wg_ref, wu_ref, w2_ref, ys_ref):
    j = pl.program_id(1)
    x = xs_ref[...]
    gate = jax.lax.dot_general(
        x, wg_ref[0], (((1,), (1,)), ((), ())),
        preferred_element_type=jnp.float32)
    up = jax.lax.dot_general(
        x, wu_ref[0], (((1,), (1,)), ((), ())),
        preferred_element_type=jnp.float32)
    h = gate * jax.lax.logistic(gate) * up
    y = jax.lax.dot_general(
        h, w2_ref[0], (((1,), (1,)), ((), ())),
        preferred_element_type=jnp.float32)

    @pl.when(j == 0)
    def _():
        ys_ref[...] = y

    @pl.when(j > 0)
    def _():
        ys_ref[...] += y


def _grouped_mlp(blk_expert, xs, ws, w2s):
    grid_spec = pltpu.PrefetchScalarGridSpec(
        num_scalar_prefetch=1,
        grid=(G, NJ),
        in_specs=[
            pl.BlockSpec((B, D), lambda g, j, be: (g, 0)),
            pl.BlockSpec((1, BI, D), lambda g, j, be: (be[g], j, 0)),
            pl.BlockSpec((1, BI, D), lambda g, j, be: (be[g], j + NJ, 0)),
            pl.BlockSpec((1, D, BI), lambda g, j, be: (be[g], 0, j)),
        ],
        out_specs=pl.BlockSpec((B, D), lambda g, j, be: (g, 0)),
    )
    return pl.pallas_call(
        _moe_body,
        grid_spec=grid_spec,
        out_shape=jax.ShapeDtypeStruct((P, D), jnp.float32),
        compiler_params=pltpu.CompilerParams(
            dimension_semantics=("arbitrary", "arbitrary")),
    )(blk_expert, xs, ws, ws, w2s)


def kernel(hidden_states, router_w, ws, w2s):
    x = hidden_states
    w_full, i_full = _router(x, router_w)
    topk_w = w_full[:, :K]
    topk_idx = i_full[:, :K]

    e_flat = topk_idx.reshape(-1)
    onehot = (e_flat[:, None] == jnp.arange(E)[None, :]).astype(jnp.int32)
    ranks = jnp.cumsum(onehot, axis=0) - 1
    counts = jnp.sum(onehot, axis=0)
    padded = ((counts + B - 1) // B) * B
    ends = jnp.cumsum(padded)
    offs = ends - padded
    rank = jnp.take_along_axis(ranks, e_flat[:, None], axis=1)[:, 0]
    pos = offs[e_flat] + rank
    tok = jnp.arange(K * T, dtype=jnp.int32) // K
    tok_idx = jnp.zeros((P,), jnp.int32).at[pos].set(tok)
    blk_expert = jnp.minimum(
        jnp.searchsorted(ends, jnp.arange(G, dtype=jnp.int32) * B, side="right"),
        E - 1).astype(jnp.int32)
    inv = pos.reshape(T, K)

    xs = jnp.take(x, tok_idx, axis=0)
    ys = _grouped_mlp(blk_expert, xs, ws, w2s)
    out = (topk_w[:, 0:1] * jnp.take(ys, inv[:, 0], axis=0)
           + topk_w[:, 1:2] * jnp.take(ys, inv[:, 1], axis=0))
    return out


# speedup vs baseline: 1.3243x; 1.1230x over previous
import functools

import jax
import jax.numpy as jnp
from jax.experimental import pallas as pl
from jax.experimental.pallas import tpu as pltpu

T = 2048
D = 1024
I = 4096
E = 8
K = 2

CAP = T
P2 = E * CAP
SB = 256
NSB = CAP // SB
BI = 512
NJ = I // BI


def _router_bin_body(x_ref, rw_ref, posi_ref, wts_ref, cnt_ref):
    logits = jax.lax.dot_general(
        x_ref[...], rw_ref[...], (((1,), (1,)), ((), ())),
        preferred_element_type=jnp.float32)
    col = jax.lax.broadcasted_iota(jnp.int32, logits.shape, 1)
    m1 = jnp.max(logits, axis=-1, keepdims=True)
    i1 = jnp.min(jnp.where(logits == m1, col, E), axis=-1, keepdims=True)
    masked = jnp.where(col == i1, -jnp.inf, logits)
    m2 = jnp.max(masked, axis=-1, keepdims=True)
    i2 = jnp.min(jnp.where(masked == m2, col, E), axis=-1, keepdims=True)
    w1 = 1.0 / (1.0 + jnp.exp(m2 - m1))
    w2 = 1.0 - w1

    onehot1 = (col == i1).astype(jnp.int32)
    onehot2 = (col == i2).astype(jnp.int32)
    m = onehot1 + onehot2
    cum = m
    for s in (1, 2, 4, 8, 16, 32, 64, 128, 256, 512, 1024):
        cum = cum + jnp.concatenate(
            [jnp.zeros((s, E), jnp.int32), cum[:T - s, :]], axis=0)
    cex = cum - m
    pos1 = i1 * CAP + jnp.sum(onehot1 * cex, axis=-1, keepdims=True)
    pos2 = i2 * CAP + jnp.sum(onehot2 * cex, axis=-1, keepdims=True)

    posi_ref[...] = jnp.where(col == 0, pos1, jnp.where(col == 1, pos2, 0))
    wts_ref[...] = jnp.where(col == 0, w1, jnp.where(col == 1, w2, 0.0))
    counts = jax.lax.dot_general(
        m.astype(jnp.float32), jnp.ones((T, 1), jnp.float32),
        (((0,), (0,)), ((), ())), preferred_element_type=jnp.float32)
    cnt_ref[...] = jnp.broadcast_to(counts.astype(jnp.int32), (E, 128))


def _router_bin(x, router_w):
    return pl.pallas_call(
        _router_bin_body,
        out_shape=(
            jax.ShapeDtypeStruct((T, E), jnp.int32),
            jax.ShapeDtypeStruct((T, E), jnp.float32),
            jax.ShapeDtypeStruct((E, 128), jnp.int32),
        ),
    )(x, router_w)


def _moe_body(cnt_ref, xs_ref, wg_ref, wu_ref, w2_ref, wr_ref, ys_ref):
    e = pl.program_id(0)
    j = pl.program_id(1)
    c = cnt_ref[e, 0]
    wgb = wg_ref[0].astype(jnp.bfloat16)
    wub = wu_ref[0].astype(jnp.bfloat16)
    w2b = w2_ref[0].astype(jnp.bfloat16)
    for sb in range(NSB):
        @pl.when(sb * SB < c)
        def _():
            rows = pl.ds(sb * SB, SB)
            xb = xs_ref[rows, :].astype(jnp.bfloat16)
            gate = jax.lax.dot_general(
                xb, wgb, (((1,), (1,)), ((), ())),
                preferred_element_type=jnp.float32)
            up = jax.lax.dot_general(
                xb, wub, (((1,), (1,)), ((), ())),
                preferred_element_type=jnp.float32)
            h = (gate * jax.lax.logistic(gate) * up).astype(jnp.bfloat16)
            y = jax.lax.dot_general(
                h, w2b, (((1,), (1,)), ((), ())),
                preferred_element_type=jnp.float32)
            yw = y * wr_ref[rows, :]

            @pl.when(j == 0)
            def _():
                ys_ref[rows, :] = yw

            @pl.when(j > 0)
            def _():
                ys_ref[rows, :] += yw


def _grouped_mlp(counts, xs, ws, w2s, wsorted):
    grid_spec = pltpu.PrefetchScalarGridSpec(
        num_scalar_prefetch=1,
        grid=(E, NJ),
        in_specs=[
            pl.BlockSpec((CAP, D), lambda e, j, c: (e, 0)),
            pl.BlockSpec((1, BI, D), lambda e, j, c: (e, j, 0)),
            pl.BlockSpec((1, BI, D), lambda e, j, c: (e, j + NJ, 0)),
            pl.BlockSpec((1, D, BI), lambda e, j, c: (e, 0, j)),
            pl.BlockSpec((CAP, 1), lambda e, j, c: (e, 0)),
        ],
        out_specs=pl.BlockSpec((CAP, D), lambda e, j, c: (e, 0)),
    )
    return pl.pallas_call(
        _moe_body,
        grid_spec=grid_spec,
        out_shape=jax.ShapeDtypeStruct((P2, D), jnp.float32),
        compiler_params=pltpu.CompilerParams(
            dimension_semantics=("arbitrary", "arbitrary")),
    )(counts, xs, ws, ws, w2s, wsorted)


def kernel(hidden_states, router_w, ws, w2s):
    x = hidden_states
    posi, wts, counts = _router_bin(x, router_w)
    pos = posi[:, :K].reshape(-1)
    wflat = wts[:, :K].reshape(-1)

    tok = jnp.arange(K * T, dtype=jnp.int32) // K
    tok_idx = jnp.zeros((P2,), jnp.int32).at[pos].set(tok)
    wsorted = jnp.zeros((P2, 1), jnp.float32).at[pos, 0].set(wflat)

    xs = jnp.take(x, tok_idx, axis=0)
    ys = _grouped_mlp(counts, xs, ws, w2s, wsorted)
    out = (jnp.take(ys, posi[:, 0], axis=0)
           + jnp.take(ys, posi[:, 1], axis=0))
    return out


# speedup vs baseline: 1.7833x; 1.3465x over previous
import functools

import jax
import jax.numpy as jnp
from jax import lax
from jax.experimental import pallas as pl
from jax.experimental.pallas import tpu as pltpu
from jax.experimental.pallas import tpu_sc as plsc

T = 2048
D = 1024
I = 4096
E = 8
K = 2

CAP = T
P2 = E * CAP
SB = 256
NSB = CAP // SB
BI = 512
NJ = I // BI


def _router_bin_body(x_ref, rw_ref, posi_ref, wts_ref, cnt_ref):
    logits = jax.lax.dot_general(
        x_ref[...], rw_ref[...], (((1,), (1,)), ((), ())),
        preferred_element_type=jnp.float32)
    col = jax.lax.broadcasted_iota(jnp.int32, logits.shape, 1)
    m1 = jnp.max(logits, axis=-1, keepdims=True)
    i1 = jnp.min(jnp.where(logits == m1, col, E), axis=-1, keepdims=True)
    masked = jnp.where(col == i1, -jnp.inf, logits)
    m2 = jnp.max(masked, axis=-1, keepdims=True)
    i2 = jnp.min(jnp.where(masked == m2, col, E), axis=-1, keepdims=True)
    w1 = 1.0 / (1.0 + jnp.exp(m2 - m1))
    w2 = 1.0 - w1

    onehot1 = (col == i1).astype(jnp.int32)
    onehot2 = (col == i2).astype(jnp.int32)
    m = onehot1 + onehot2
    cum = m
    for s in (1, 2, 4, 8, 16, 32, 64, 128, 256, 512, 1024):
        cum = cum + jnp.concatenate(
            [jnp.zeros((s, E), jnp.int32), cum[:T - s, :]], axis=0)
    cex = cum - m
    pos1 = i1 * CAP + jnp.sum(onehot1 * cex, axis=-1, keepdims=True)
    pos2 = i2 * CAP + jnp.sum(onehot2 * cex, axis=-1, keepdims=True)

    posi_ref[...] = jnp.where(col == 0, pos1, jnp.where(col == 1, pos2, 0))
    wts_ref[...] = jnp.where(col == 0, w1, jnp.where(col == 1, w2, 0.0))
    counts = jax.lax.dot_general(
        m.astype(jnp.float32), jnp.ones((T, 1), jnp.float32),
        (((0,), (0,)), ((), ())), preferred_element_type=jnp.float32)
    cnt_ref[...] = jnp.broadcast_to(counts.astype(jnp.int32), (E, 128))


def _router_bin(x, router_w):
    return pl.pallas_call(
        _router_bin_body,
        out_shape=(
            jax.ShapeDtypeStruct((T, E), jnp.int32),
            jax.ShapeDtypeStruct((T, E), jnp.float32),
            jax.ShapeDtypeStruct((E, 128), jnp.int32),
        ),
    )(x, router_w)


def _moe_body(cnt_ref, xs_ref, wg_ref, wu_ref, w2_ref, wr_ref, ys_ref):
    e = pl.program_id(0)
    j = pl.program_id(1)
    c = cnt_ref[e, 0]
    wgb = wg_ref[0].astype(jnp.bfloat16)
    wub = wu_ref[0].astype(jnp.bfloat16)
    w2b = w2_ref[0].astype(jnp.bfloat16)
    for sb in range(NSB):
        @pl.when(sb * SB < c)
        def _():
            rows = pl.ds(sb * SB, SB)
            xb = xs_ref[rows, :].astype(jnp.bfloat16)
            gate = jax.lax.dot_general(
                xb, wgb, (((1,), (1,)), ((), ())),
                preferred_element_type=jnp.float32)
            up = jax.lax.dot_general(
                xb, wub, (((1,), (1,)), ((), ())),
                preferred_element_type=jnp.float32)
            h = (gate * jax.lax.logistic(gate) * up).astype(jnp.bfloat16)
            y = jax.lax.dot_general(
                h, w2b, (((1,), (1,)), ((), ())),
                preferred_element_type=jnp.float32)
            yw = y * wr_ref[rows, :]

            @pl.when(j == 0)
            def _():
                ys_ref[rows, :] = yw

            @pl.when(j > 0)
            def _():
                ys_ref[rows, :] += yw


def _grouped_mlp(counts, xs, ws, w2s, wsorted):
    grid_spec = pltpu.PrefetchScalarGridSpec(
        num_scalar_prefetch=1,
        grid=(E, NJ),
        in_specs=[
            pl.BlockSpec((CAP, D), lambda e, j, c: (e, 0)),
            pl.BlockSpec((1, BI, D), lambda e, j, c: (e, j, 0)),
            pl.BlockSpec((1, BI, D), lambda e, j, c: (e, j + NJ, 0)),
            pl.BlockSpec((1, D, BI), lambda e, j, c: (e, 0, j)),
            pl.BlockSpec((CAP, 1), lambda e, j, c: (e, 0)),
        ],
        out_specs=pl.BlockSpec((CAP, D), lambda e, j, c: (e, 0)),
    )
    return pl.pallas_call(
        _moe_body,
        grid_spec=grid_spec,
        out_shape=jax.ShapeDtypeStruct((P2, D), jnp.float32),
        compiler_params=pltpu.CompilerParams(
            dimension_semantics=("arbitrary", "arbitrary")),
    )(counts, xs, ws, ws, w2s, wsorted)


NC = 2
NS = 16
NW = NC * NS
SLOTS_W = P2 // NW
TOK_W = T // NW
RC = 64
RCC = 32

_sc_mesh = plsc.VectorSubcoreMesh(
    core_axis_name="c", subcore_axis_name="s", num_cores=NC, num_subcores=NS)


def _dispatch_body(x_hbm, pos_hbm, wts_hbm, xs_hbm, wr_hbm,
                   xrows_v, idx0_v, idx1_v, w0_v, w1_v, sem):
    wid = lax.axis_index("s") * NC + lax.axis_index("c")
    t0 = wid * TOK_W
    a0 = pltpu.async_copy(pos_hbm.at[pl.ds(t0, TOK_W)], idx0_v, sem)
    a1 = pltpu.async_copy(pos_hbm.at[pl.ds(T + t0, TOK_W)], idx1_v, sem)
    a2 = pltpu.async_copy(wts_hbm.at[pl.ds(t0, TOK_W)], w0_v, sem)
    a3 = pltpu.async_copy(wts_hbm.at[pl.ds(T + t0, TOK_W)], w1_v, sem)
    a4 = pltpu.async_copy(x_hbm.at[pl.ds(t0, TOK_W)], xrows_v, sem)
    a0.wait()
    a1.wait()
    a2.wait()
    a3.wait()
    a4.wait()
    c0 = pltpu.async_copy(xrows_v, xs_hbm.at[idx0_v], sem)
    c1 = pltpu.async_copy(xrows_v, xs_hbm.at[idx1_v], sem)
    c2 = pltpu.async_copy(w0_v, wr_hbm.at[idx0_v], sem)
    c3 = pltpu.async_copy(w1_v, wr_hbm.at[idx1_v], sem)
    c0.wait()
    c1.wait()
    c2.wait()
    c3.wait()


def _dispatch(x, pos_flat, wts_flat):
    return pl.kernel(
        _dispatch_body,
        out_type=(
            jax.ShapeDtypeStruct((P2, D), jnp.float32),
            jax.ShapeDtypeStruct((P2,), jnp.float32),
        ),
        mesh=_sc_mesh,
        scratch_types=[
            pltpu.VMEM((TOK_W, D), jnp.float32),
            pltpu.VMEM((TOK_W,), jnp.int32),
            pltpu.VMEM((TOK_W,), jnp.int32),
            pltpu.VMEM((TOK_W,), jnp.float32),
            pltpu.VMEM((TOK_W,), jnp.float32),
            pltpu.SemaphoreType.DMA,
        ],
    )(x, pos_flat, wts_flat)


def _combine_body(ys_hbm, pos_hbm, out_hbm,
                  idx0_v, idx1_v, r0_v, r1_v, sem):
    wid = lax.axis_index("s") * NC + lax.axis_index("c")
    t0 = wid * TOK_W
    for ch in range(TOK_W // RCC):
        b = t0 + ch * RCC
        pltpu.sync_copy(pos_hbm.at[pl.ds(b, RCC)], idx0_v)
        pltpu.sync_copy(pos_hbm.at[pl.ds(T + b, RCC)], idx1_v)
        c0 = pltpu.async_copy(ys_hbm.at[idx0_v], r0_v, sem)
        c1 = pltpu.async_copy(ys_hbm.at[idx1_v], r1_v, sem)
        c0.wait()
        c1.wait()

        def addrow(r, carry):
            for cc in range(D // 16):
                sl = pl.ds(cc * 16, 16)
                r0_v[r, sl] += r1_v[r, sl]
            return carry

        lax.fori_loop(0, RCC, addrow, 0)
        pltpu.sync_copy(r0_v, out_hbm.at[pl.ds(b, RCC)])


def _combine(ys, pos_flat):
    return pl.kernel(
        _combine_body,
        out_type=jax.ShapeDtypeStruct((T, D), jnp.float32),
        mesh=_sc_mesh,
        scratch_types=[
            pltpu.VMEM((RCC,), jnp.int32),
            pltpu.VMEM((RCC,), jnp.int32),
            pltpu.VMEM((RCC, D), jnp.float32),
            pltpu.VMEM((RCC, D), jnp.float32),
            pltpu.SemaphoreType.DMA,
        ],
    )(ys, pos_flat)


def kernel(hidden_states, router_w, ws, w2s):
    x = hidden_states
    posi, wts, counts = _router_bin(x, router_w)
    pos_flat = jnp.concatenate([posi[:, 0], posi[:, 1]])
    wts_flat = jnp.concatenate([wts[:, 0], wts[:, 1]])
    xs, wsorted = _dispatch(x, pos_flat, wts_flat)
    ys = _grouped_mlp(counts, xs, ws, w2s, wsorted.reshape(P2, 1))
    return _combine(ys, pos_flat)


# speedup vs baseline: 1.7841x; 1.0004x over previous
import jax
import jax.numpy as jnp
from jax import lax
from jax.experimental import pallas as pl
from jax.experimental.pallas import tpu as pltpu
from jax.experimental.pallas import tpu_sc as plsc

T = 2048
D = 1024
I = 4096
E = 8
K = 2

CAP = T
P2 = E * CAP
SB = 256
NSB = CAP // SB
BI = 512
NJ = I // BI


def _router_bin_body(x_ref, rw_ref, posi_ref, wts_ref, cnt_ref):
    logits = jax.lax.dot_general(
        x_ref[...], rw_ref[...], (((1,), (1,)), ((), ())),
        preferred_element_type=jnp.float32)
    col = jax.lax.broadcasted_iota(jnp.int32, logits.shape, 1)
    m1 = jnp.max(logits, axis=-1, keepdims=True)
    i1 = jnp.min(jnp.where(logits == m1, col, E), axis=-1, keepdims=True)
    masked = jnp.where(col == i1, -jnp.inf, logits)
    m2 = jnp.max(masked, axis=-1, keepdims=True)
    i2 = jnp.min(jnp.where(masked == m2, col, E), axis=-1, keepdims=True)
    w1 = 1.0 / (1.0 + jnp.exp(m2 - m1))
    w2 = 1.0 - w1

    onehot1 = (col == i1).astype(jnp.int32)
    onehot2 = (col == i2).astype(jnp.int32)
    m = onehot1 + onehot2
    cum = m
    for s in (1, 2, 4, 8, 16, 32, 64, 128, 256, 512, 1024):
        cum = cum + jnp.concatenate(
            [jnp.zeros((s, E), jnp.int32), cum[:T - s, :]], axis=0)
    cex = cum - m
    pos1 = i1 * CAP + jnp.sum(onehot1 * cex, axis=-1, keepdims=True)
    pos2 = i2 * CAP + jnp.sum(onehot2 * cex, axis=-1, keepdims=True)

    posi_ref[...] = jnp.where(col == 0, pos1, jnp.where(col == 1, pos2, 0))
    wts_ref[...] = jnp.where(col == 0, w1, jnp.where(col == 1, w2, 0.0))
    counts = jax.lax.dot_general(
        m.astype(jnp.float32), jnp.ones((T, 1), jnp.float32),
        (((0,), (0,)), ((), ())), preferred_element_type=jnp.float32)
    cnt_ref[...] = jnp.broadcast_to(counts.astype(jnp.int32), (E, 128))


def _router_bin(x, router_w):
    return pl.pallas_call(
        _router_bin_body,
        out_shape=(
            jax.ShapeDtypeStruct((T, E), jnp.int32),
            jax.ShapeDtypeStruct((T, E), jnp.float32),
            jax.ShapeDtypeStruct((E, 128), jnp.int32),
        ),
    )(x, router_w)


def _moe_body(cnt_ref, xs_ref, wg_ref, wu_ref, w2_ref, wr_ref, ys_ref):
    e = pl.program_id(0)
    j = pl.program_id(1)
    c = cnt_ref[e, 0]
    wgb = wg_ref[0].astype(jnp.bfloat16)
    wub = wu_ref[0].astype(jnp.bfloat16)
    w2b = w2_ref[0].astype(jnp.bfloat16)
    for sb in range(NSB):
        @pl.when(sb * SB < c)
        def _():
            rows = pl.ds(sb * SB, SB)
            xb = xs_ref[rows, :].astype(jnp.bfloat16)
            gate = jax.lax.dot_general(
                xb, wgb, (((1,), (1,)), ((), ())),
                preferred_element_type=jnp.float32)
            up = jax.lax.dot_general(
                xb, wub, (((1,), (1,)), ((), ())),
                preferred_element_type=jnp.float32)
            h = (gate * jax.lax.logistic(gate) * up).astype(jnp.bfloat16)
            y = jax.lax.dot_general(
                h, w2b, (((1,), (1,)), ((), ())),
                preferred_element_type=jnp.float32)
            yw = y * wr_ref[rows, :]

            @pl.when(j == 0)
            def _():
                ys_ref[rows, :] = yw

            @pl.when(j > 0)
            def _():
                ys_ref[rows, :] += yw


def _grouped_mlp(counts, xs, ws, w2s, wsorted):
    grid_spec = pltpu.PrefetchScalarGridSpec(
        num_scalar_prefetch=1,
        grid=(E, NJ),
        in_specs=[
            pl.BlockSpec((CAP, D), lambda e, j, c: (e, 0)),
            pl.BlockSpec((1, BI, D), lambda e, j, c: (e, j, 0)),
            pl.BlockSpec((1, BI, D), lambda e, j, c: (e, j + NJ, 0)),
            pl.BlockSpec((1, D, BI), lambda e, j, c: (e, 0, j)),
            pl.BlockSpec((CAP, 1), lambda e, j, c: (e, 0)),
        ],
        out_specs=pl.BlockSpec((CAP, D), lambda e, j, c: (e, 0)),
    )
    return pl.pallas_call(
        _moe_body,
        grid_spec=grid_spec,
        out_shape=jax.ShapeDtypeStruct((P2, D), jnp.float32),
        compiler_params=pltpu.CompilerParams(
            dimension_semantics=("arbitrary", "arbitrary")),
    )(counts, xs, ws, ws, w2s, wsorted)


NC = 2
NS = 16
NW = NC * NS
SLOTS_W = P2 // NW
TOK_W = T // NW
RC = 64
RCC = 32

_sc_mesh = plsc.VectorSubcoreMesh(
    core_axis_name="c", subcore_axis_name="s", num_cores=NC, num_subcores=NS)


def _dispatch_body(x_hbm, pos_hbm, wts_hbm, xs_hbm, wr_hbm,
                   xrows_v, idx0_v, idx1_v, w0_v, w1_v, sem):
    wid = lax.axis_index("s") * NC + lax.axis_index("c")
    t0 = wid * TOK_W
    a0 = pltpu.async_copy(pos_hbm.at[pl.ds(t0, TOK_W)], idx0_v, sem)
    a1 = pltpu.async_copy(pos_hbm.at[pl.ds(T + t0, TOK_W)], idx1_v, sem)
    a2 = pltpu.async_copy(wts_hbm.at[pl.ds(t0, TOK_W)], w0_v, sem)
    a3 = pltpu.async_copy(wts_hbm.at[pl.ds(T + t0, TOK_W)], w1_v, sem)
    a4 = pltpu.async_copy(x_hbm.at[pl.ds(t0, TOK_W)], xrows_v, sem)
    a0.wait()
    a1.wait()
    a2.wait()
    a3.wait()
    a4.wait()
    c0 = pltpu.async_copy(xrows_v, xs_hbm.at[idx0_v], sem)
    c1 = pltpu.async_copy(xrows_v, xs_hbm.at[idx1_v], sem)
    c2 = pltpu.async_copy(w0_v, wr_hbm.at[idx0_v], sem)
    c3 = pltpu.async_copy(w1_v, wr_hbm.at[idx1_v], sem)
    c0.wait()
    c1.wait()
    c2.wait()
    c3.wait()


def _dispatch(x, pos_flat, wts_flat):
    return pl.kernel(
        _dispatch_body,
        out_type=(
            jax.ShapeDtypeStruct((P2, D), jnp.float32),
            jax.ShapeDtypeStruct((P2,), jnp.float32),
        ),
        mesh=_sc_mesh,
        scratch_types=[
            pltpu.VMEM((TOK_W, D), jnp.float32),
            pltpu.VMEM((TOK_W,), jnp.int32),
            pltpu.VMEM((TOK_W,), jnp.int32),
            pltpu.VMEM((TOK_W,), jnp.float32),
            pltpu.VMEM((TOK_W,), jnp.float32),
            pltpu.SemaphoreType.DMA,
        ],
    )(x, pos_flat, wts_flat)


def _combine_body(ys_hbm, pos_hbm, out_hbm,
                  idx0_v, idx1_v, r0_v, r1_v, sem):
    wid = lax.axis_index("s") * NC + lax.axis_index("c")
    t0 = wid * TOK_W
    for ch in range(TOK_W // RCC):
        b = t0 + ch * RCC
        pltpu.sync_copy(pos_hbm.at[pl.ds(b, RCC)], idx0_v)
        pltpu.sync_copy(pos_hbm.at[pl.ds(T + b, RCC)], idx1_v)
        c0 = pltpu.async_copy(ys_hbm.at[idx0_v], r0_v, sem)
        c1 = pltpu.async_copy(ys_hbm.at[idx1_v], r1_v, sem)
        c0.wait()
        c1.wait()

        def addrow(r, carry):
            for cc in range(D // 16):
                sl = pl.ds(cc * 16, 16)
                r0_v[r, sl] += r1_v[r, sl]
            return carry

        lax.fori_loop(0, RCC, addrow, 0)
        pltpu.sync_copy(r0_v, out_hbm.at[pl.ds(b, RCC)])


def _combine(ys, pos_flat):
    return pl.kernel(
        _combine_body,
        out_type=jax.ShapeDtypeStruct((T, D), jnp.float32),
        mesh=_sc_mesh,
        scratch_types=[
            pltpu.VMEM((RCC,), jnp.int32),
            pltpu.VMEM((RCC,), jnp.int32),
            pltpu.VMEM((RCC, D), jnp.float32),
            pltpu.VMEM((RCC, D), jnp.float32),
            pltpu.SemaphoreType.DMA,
        ],
    )(ys, pos_flat)


def kernel(hidden_states, router_w, ws, w2s):
    x = hidden_states
    posi, wts, counts = _router_bin(x, router_w)
    pos_flat = jnp.concatenate([posi[:, 0], posi[:, 1]])
    wts_flat = jnp.concatenate([wts[:, 0], wts[:, 1]])
    xs, wsorted = _dispatch(x, pos_flat, wts_flat)
    ys = _grouped_mlp(counts, xs, ws, w2s, wsorted.reshape(P2, 1))
    return _combine(ys, pos_flat)
